# baseline (speedup 1.0000x reference)
import functools

import jax
import jax.numpy as jnp
from jax import lax
from jax.experimental import pallas as pl
from jax.experimental.pallas import tpu as pltpu
from jax.experimental.pallas import tpu_sc as plsc

NUM_FIELDS = 26
FIELD_DIM = 100000
EMBED_DIM = 16
BATCH = 16384

L = 16
NC = 2
NS = 16
NW = NC * NS

B = BATCH * NUM_FIELDS
BPW = B // NW
CHUNK = 3328
NCH = BPW // CHUNK


@functools.partial(
    pl.kernel,
    mesh=plsc.VectorSubcoreMesh(core_axis_name="c", subcore_axis_name="s"),
    compiler_params=pltpu.CompilerParams(use_tc_tiling_on_sc=False),
    out_type=jax.ShapeDtypeStruct((B, EMBED_DIM), jnp.float32),
    scratch_types=[
        pltpu.VMEM((BPW,), jnp.int32),
        pltpu.VMEM((CHUNK, EMBED_DIM), jnp.float32),
        pltpu.VMEM((CHUNK, EMBED_DIM), jnp.float32),
        pltpu.SemaphoreType.DMA,
        pltpu.SemaphoreType.DMA,
    ],
)
def _emb_gather(x_hbm, table_hbm, out_hbm, idx_v, rows0, rows1, sem0, sem1):
    wid = lax.axis_index("s") * NC + lax.axis_index("c")
    base = wid * BPW

    pltpu.sync_copy(x_hbm.at[pl.ds(base, BPW)], idx_v)

    lane = lax.iota(jnp.int32, 16)

    def add_offsets(i, carry):
        pos = base + i * L + lane
        idx_v[pl.ds(i * L, L)] = idx_v[pl.ds(i * L, L)] + (pos % NUM_FIELDS) * FIELD_DIM
        return carry

    lax.fori_loop(0, BPW // L, add_offsets, 0)

    rows = (rows0, rows1)
    sems = (sem0, sem1)
    copies = []
    for c in range(NCH):
        cp = pltpu.async_copy(
            table_hbm.at[idx_v.at[pl.ds(c * CHUNK, CHUNK)]], rows[c % 2], sems[c % 2]
        )
        copies.append(cp)
        if c >= 1:
            copies[c - 1].wait()
            pltpu.sync_copy(
                rows[(c - 1) % 2], out_hbm.at[pl.ds(base + (c - 1) * CHUNK, CHUNK)]
            )
    copies[NCH - 1].wait()
    pltpu.sync_copy(
        rows[(NCH - 1) % 2], out_hbm.at[pl.ds(base + (NCH - 1) * CHUNK, CHUNK)]
    )


def kernel(x, table):
    flat = x.reshape(B)
    out = _emb_gather(flat, table)
    return out.reshape(BATCH, NUM_FIELDS, EMBED_DIM)
